# baseline (speedup 1.0000x reference)
import functools

import jax
import jax.numpy as jnp
from jax import lax
from jax.experimental import pallas as pl
from jax.experimental.pallas import tpu as pltpu
from jax.experimental.pallas import tpu_sc as plsc

B_ROWS = 16384
B_COLS = 26
DIM = 64

_info = plsc.get_sparse_core_info()
NC = _info.num_cores
NS = _info.num_subcores
NW = NC * NS
ROWS_PER_W = B_ROWS // NW
RCHUNK = 16
N_CHUNKS = ROWS_PER_W // RCHUNK
NBUF = 2

_mesh = plsc.VectorSubcoreMesh(core_axis_name="c", subcore_axis_name="s")


@functools.partial(
    pl.kernel,
    out_type=jax.ShapeDtypeStruct((B_ROWS, B_COLS, DIM), jnp.float32),
    mesh=_mesh,
    scratch_types=[
        pltpu.VMEM((RCHUNK, B_COLS), jnp.int32),
        pltpu.VMEM((RCHUNK, B_COLS), jnp.int32),
        pltpu.VMEM((RCHUNK, B_COLS, DIM), jnp.float32),
        pltpu.VMEM((RCHUNK, B_COLS, DIM), jnp.float32),
        pltpu.SemaphoreType.DMA,
        pltpu.SemaphoreType.DMA,
        pltpu.SemaphoreType.DMA,
        pltpu.SemaphoreType.DMA,
    ],
    compiler_params=pltpu.CompilerParams(use_tc_tiling_on_sc=False),
)
def _gather_kernel(x_hbm, tab_hbm, out_hbm, idx0, idx1, rows0, rows1,
                   gsem0, gsem1, osem0, osem1):
    wid = lax.axis_index("s") * NC + lax.axis_index("c")
    base = wid * ROWS_PER_W
    bufs = ((idx0, rows0, gsem0, osem0), (idx1, rows1, gsem1, osem1))

    def fire_gathers(idx_v, rows_v, gsem):
        for r in range(RCHUNK):
            pltpu.async_copy(tab_hbm.at[idx_v.at[r]], rows_v.at[r], gsem)

    def drain_gathers(rows_v, gsem, row0):
        pltpu.make_async_copy(
            out_hbm.at[pl.ds(row0, RCHUNK)], rows_v, gsem).wait()

    for b in range(NBUF):
        idx_v, rows_v, gsem, _ = bufs[b]
        row0 = base + b * RCHUNK
        pltpu.sync_copy(x_hbm.at[pl.ds(row0, RCHUNK)], idx_v)
        fire_gathers(idx_v, rows_v, gsem)

    def outer(j, carry):
        for b in range(NBUF):
            i = j * NBUF + b
            idx_v, rows_v, gsem, osem = bufs[b]
            row0 = base + i * RCHUNK
            drain_gathers(rows_v, gsem, row0)
            pltpu.async_copy(rows_v, out_hbm.at[pl.ds(row0, RCHUNK)], osem)

            @pl.when(i + NBUF < N_CHUNKS)
            def _():
                row2 = base + (i + NBUF) * RCHUNK
                pltpu.sync_copy(x_hbm.at[pl.ds(row2, RCHUNK)], idx_v)
                pltpu.make_async_copy(
                    rows_v, out_hbm.at[pl.ds(row0, RCHUNK)], osem).wait()
                fire_gathers(idx_v, rows_v, gsem)

            @pl.when(i + NBUF >= N_CHUNKS)
            def _():
                pltpu.make_async_copy(
                    rows_v, out_hbm.at[pl.ds(row0, RCHUNK)], osem).wait()

        return carry

    lax.fori_loop(0, N_CHUNKS // NBUF, outer, 0)


def kernel(x, emb_weight):
    return _gather_kernel(x, emb_weight)
